# baseline (speedup 1.0000x reference)
import jax
import jax.numpy as jnp
from jax.experimental import pallas as pl
from jax.experimental.pallas import tpu as pltpu
from functools import partial

COMMIT_W = 0.25


def _vq_body(x_ref, eT_ref, zq_ref, idx_ref, loss_ref, perp_ref, util_ref,
             e2_ref, cnt_ref, *, n_rows, k_codes, n_blocks):
    r = pl.program_id(0)

    @pl.when(r == 0)
    def _init():
        eT0 = eT_ref[...]
        e2_ref[...] = jnp.sum(eT0 * eT0, axis=0, keepdims=True)
        cnt_ref[...] = jnp.zeros_like(cnt_ref)
        loss_ref[...] = jnp.zeros_like(loss_ref)

    x = x_ref[...]
    raw = jax.lax.dot_general(
        x, eT_ref[...], (((1,), (0,)), ((), ())),
        precision=jax.lax.Precision.DEFAULT,
        preferred_element_type=jnp.float32)
    scores = e2_ref[...] - 2.0 * raw
    m = jnp.min(scores, axis=1, keepdims=True)
    iota = jax.lax.broadcasted_iota(jnp.int32, scores.shape, 1)
    idxcol = jnp.min(jnp.where(scores == m, iota, k_codes),
                     axis=1, keepdims=True)
    idx_ref[...] = idxcol

    onehot = (iota == idxcol).astype(jnp.float32)
    cnt_ref[...] += jnp.sum(onehot, axis=0, keepdims=True)
    zq = jax.lax.dot_general(
        onehot, eT_ref[...], (((1,), (1,)), ((), ())),
        precision=jax.lax.Precision.HIGHEST,
        preferred_element_type=jnp.float32)
    diff = zq - x
    zq_ref[...] = x + diff
    loss_ref[...] += jnp.sum(diff * diff, keepdims=True).reshape(1, 1)

    @pl.when(r == n_blocks - 1)
    def _finalize():
        p = cnt_ref[...] / n_rows + 1e-10
        perp_ref[...] = jnp.exp(
            -jnp.sum(p * jnp.log(p), keepdims=True).reshape(1, 1))
        util_ref[...] = jnp.sum((p > 1e-8).astype(jnp.float32),
                                keepdims=True).reshape(1, 1) / k_codes
        d = x_ref[...].shape[-1]
        loss_ref[...] = loss_ref[...] * ((1.0 + COMMIT_W) / (n_rows * d))


def kernel(z_e, embed):
    B, M, D = z_e.shape
    N = B * M
    K = embed.shape[0]
    x = z_e.reshape(N, D)
    eT = embed.T

    BR = min(256, N)
    G = N // BR

    out_shape = (
        jax.ShapeDtypeStruct((N, D), jnp.float32),
        jax.ShapeDtypeStruct((N, 1), jnp.int32),
        jax.ShapeDtypeStruct((1, 1), jnp.float32),
        jax.ShapeDtypeStruct((1, 1), jnp.float32),
        jax.ShapeDtypeStruct((1, 1), jnp.float32),
    )
    scalar_spec = pl.BlockSpec((1, 1), lambda r: (0, 0))
    zq_flat, idx_col, loss, perp, util = pl.pallas_call(
        partial(_vq_body, n_rows=N, k_codes=K, n_blocks=G),
        grid=(G,),
        in_specs=[
            pl.BlockSpec((BR, D), lambda r: (r, 0)),
            pl.BlockSpec((D, K), lambda r: (0, 0)),
        ],
        out_specs=(
            pl.BlockSpec((BR, D), lambda r: (r, 0)),
            pl.BlockSpec((BR, 1), lambda r: (r, 0)),
            scalar_spec, scalar_spec, scalar_spec,
        ),
        out_shape=out_shape,
        scratch_shapes=[
            pltpu.VMEM((1, K), jnp.float32),
            pltpu.VMEM((1, K), jnp.float32),
        ],
        compiler_params=pltpu.CompilerParams(
            dimension_semantics=("arbitrary",)),
    )(x, eT)

    z_q_st = zq_flat.reshape(B, M, D)
    indices = idx_col.reshape(B, M)
    return (z_q_st, indices, loss[0, 0], perp[0, 0], util[0, 0])


# speedup vs baseline: 1.8716x; 1.8716x over previous
import jax
import jax.numpy as jnp
from jax.experimental import pallas as pl
from jax.experimental.pallas import tpu as pltpu
from functools import partial

COMMIT_W = 0.25


def _vq_body(x_ref, eT_ref, zq_ref, idx_ref, loss_ref, perp_ref, util_ref,
             e2_ref, cnt_ref, *, n_rows, k_codes, n_blocks):
    r = pl.program_id(0)

    @pl.when(r == 0)
    def _init():
        eT0 = eT_ref[...]
        e2_ref[...] = jnp.sum(eT0 * eT0, axis=0, keepdims=True)
        cnt_ref[...] = jnp.zeros_like(cnt_ref)
        loss_ref[...] = jnp.zeros_like(loss_ref)

    x = x_ref[...]
    raw = jax.lax.dot_general(
        x, eT_ref[...], (((1,), (0,)), ((), ())),
        precision=jax.lax.Precision.DEFAULT,
        preferred_element_type=jnp.float32)
    scores = e2_ref[...] - 2.0 * raw
    m = jnp.min(scores, axis=1, keepdims=True)
    iota = jax.lax.broadcasted_iota(jnp.int32, scores.shape, 1)
    idxcol = jnp.min(jnp.where(scores == m, iota, k_codes),
                     axis=1, keepdims=True)
    idx_ref[...] = idxcol

    onehot = (iota == idxcol).astype(jnp.float32)
    cnt_ref[...] += jnp.sum(onehot, axis=0, keepdims=True)
    zq = jax.lax.dot_general(
        onehot, eT_ref[...], (((1,), (1,)), ((), ())),
        precision=jax.lax.Precision.DEFAULT,
        preferred_element_type=jnp.float32)
    diff = zq - x
    zq_ref[...] = x + diff
    loss_ref[...] += jnp.sum(diff * diff, keepdims=True).reshape(1, 1)

    @pl.when(r == n_blocks - 1)
    def _finalize():
        p = cnt_ref[...] / n_rows + 1e-10
        perp_ref[...] = jnp.exp(
            -jnp.sum(p * jnp.log(p), keepdims=True).reshape(1, 1))
        util_ref[...] = jnp.sum((p > 1e-8).astype(jnp.float32),
                                keepdims=True).reshape(1, 1) / k_codes
        d = x_ref[...].shape[-1]
        loss_ref[...] = loss_ref[...] * ((1.0 + COMMIT_W) / (n_rows * d))


def kernel(z_e, embed):
    B, M, D = z_e.shape
    N = B * M
    K = embed.shape[0]
    x = z_e.reshape(N, D)
    eT = embed.T

    BR = min(256, N)
    G = N // BR

    out_shape = (
        jax.ShapeDtypeStruct((N, D), jnp.float32),
        jax.ShapeDtypeStruct((N, 1), jnp.int32),
        jax.ShapeDtypeStruct((1, 1), jnp.float32),
        jax.ShapeDtypeStruct((1, 1), jnp.float32),
        jax.ShapeDtypeStruct((1, 1), jnp.float32),
    )
    scalar_spec = pl.BlockSpec((1, 1), lambda r: (0, 0))
    zq_flat, idx_col, loss, perp, util = pl.pallas_call(
        partial(_vq_body, n_rows=N, k_codes=K, n_blocks=G),
        grid=(G,),
        in_specs=[
            pl.BlockSpec((BR, D), lambda r: (r, 0)),
            pl.BlockSpec((D, K), lambda r: (0, 0)),
        ],
        out_specs=(
            pl.BlockSpec((BR, D), lambda r: (r, 0)),
            pl.BlockSpec((BR, 1), lambda r: (r, 0)),
            scalar_spec, scalar_spec, scalar_spec,
        ),
        out_shape=out_shape,
        scratch_shapes=[
            pltpu.VMEM((1, K), jnp.float32),
            pltpu.VMEM((1, K), jnp.float32),
        ],
        compiler_params=pltpu.CompilerParams(
            dimension_semantics=("arbitrary",)),
    )(x, eT)

    z_q_st = zq_flat.reshape(B, M, D)
    indices = idx_col.reshape(B, M)
    return (z_q_st, indices, loss[0, 0], perp[0, 0], util[0, 0])


# speedup vs baseline: 2.3041x; 1.2311x over previous
import functools

import jax
import jax.numpy as jnp
from jax import lax
from jax.experimental import pallas as pl
from jax.experimental.pallas import tpu as pltpu
from jax.experimental.pallas import tpu_sc as plsc

COMMIT_W = 0.25


def _argmin_body(x_ref, eTm2_ref, idx_ref, perp_ref, util_ref,
                 e2_ref, cnt_ref, *, n_rows, k_codes, n_blocks):
    r = pl.program_id(0)

    @pl.when(r == 0)
    def _init():
        eTm2 = eTm2_ref[...]
        e2_ref[...] = 0.25 * jnp.sum(eTm2 * eTm2, axis=0, keepdims=True)
        cnt_ref[...] = jnp.zeros_like(cnt_ref)

    x = x_ref[...]
    raw = lax.dot_general(
        x, eTm2_ref[...], (((1,), (0,)), ((), ())),
        precision=lax.Precision.DEFAULT,
        preferred_element_type=jnp.float32)
    scores = raw + e2_ref[...]
    m = jnp.min(scores, axis=1, keepdims=True)
    iota = lax.broadcasted_iota(jnp.int32, scores.shape, 1)
    idxcol = jnp.min(jnp.where(scores == m, iota, k_codes),
                     axis=1, keepdims=True)
    idx_ref[...] = idxcol
    cnt_ref[...] += jnp.sum((iota == idxcol).astype(jnp.float32),
                            axis=0, keepdims=True)

    @pl.when(r == n_blocks - 1)
    def _finalize():
        p = cnt_ref[...] / n_rows + 1e-10
        perp_ref[...] = jnp.exp(
            -jnp.sum(p * jnp.log(p), keepdims=True).reshape(1, 1))
        util_ref[...] = jnp.sum((p > 1e-8).astype(jnp.float32),
                                keepdims=True).reshape(1, 1) / k_codes


def _argmin_call(x, eTm2, n_rows, d, k_codes):
    BR = 256
    G = n_rows // BR
    scalar_spec = pl.BlockSpec((1, 1), lambda r: (0, 0))
    return pl.pallas_call(
        functools.partial(_argmin_body, n_rows=n_rows, k_codes=k_codes,
                          n_blocks=G),
        grid=(G,),
        in_specs=[
            pl.BlockSpec((BR, d), lambda r: (r, 0)),
            pl.BlockSpec((d, k_codes), lambda r: (0, 0)),
        ],
        out_specs=(
            pl.BlockSpec((BR, 1), lambda r: (r, 0)),
            scalar_spec, scalar_spec,
        ),
        out_shape=(
            jax.ShapeDtypeStruct((n_rows, 1), jnp.int32),
            jax.ShapeDtypeStruct((1, 1), jnp.float32),
            jax.ShapeDtypeStruct((1, 1), jnp.float32),
        ),
        scratch_shapes=[
            pltpu.VMEM((1, k_codes), jnp.float32),
            pltpu.VMEM((1, k_codes), jnp.float32),
        ],
        compiler_params=pltpu.CompilerParams(
            dimension_semantics=("arbitrary",)),
    )(x, eTm2)


def _make_sc_gather(n_rows, d, chunk):
    info = plsc.get_sparse_core_info()
    nc, ns = info.num_cores, info.num_subcores
    nw = nc * ns
    rows_per_w = n_rows // nw
    n_chunks = rows_per_w // chunk
    mesh = plsc.VectorSubcoreMesh(core_axis_name="c", subcore_axis_name="s")

    @functools.partial(
        pl.kernel, mesh=mesh,
        out_type=jax.ShapeDtypeStruct((n_rows, d), jnp.float32),
        scratch_types=[
            pltpu.VMEM((chunk,), jnp.int32),
            pltpu.VMEM((chunk, d), jnp.float32),
            pltpu.SemaphoreType.DMA,
        ],
    )
    def gather(table_hbm, idx_hbm, out_hbm, idx_v, rows_v, sem):
        wid = lax.axis_index("s") * nc + lax.axis_index("c")
        base = wid * rows_per_w
        for c in range(n_chunks):
            off = base + c * chunk
            pltpu.sync_copy(idx_hbm.at[pl.ds(off, chunk)], idx_v)
            pltpu.async_copy(table_hbm.at[idx_v], rows_v, sem).wait()
            pltpu.sync_copy(rows_v, out_hbm.at[pl.ds(off, chunk)])

    return gather


def _loss_body(x_ref, zq_ref, loss_ref, *, n_blocks, scale):
    r = pl.program_id(0)

    @pl.when(r == 0)
    def _init():
        loss_ref[...] = jnp.zeros_like(loss_ref)

    diff = zq_ref[...] - x_ref[...]
    loss_ref[...] += jnp.sum(diff * diff, keepdims=True).reshape(1, 1)

    @pl.when(r == n_blocks - 1)
    def _finalize():
        loss_ref[...] = loss_ref[...] * scale


def _loss_call(x, zq, n_rows, d):
    BR = 1024
    G = n_rows // BR
    scale = (1.0 + COMMIT_W) / (n_rows * d)
    return pl.pallas_call(
        functools.partial(_loss_body, n_blocks=G, scale=scale),
        grid=(G,),
        in_specs=[
            pl.BlockSpec((BR, d), lambda r: (r, 0)),
            pl.BlockSpec((BR, d), lambda r: (r, 0)),
        ],
        out_specs=pl.BlockSpec((1, 1), lambda r: (0, 0)),
        out_shape=jax.ShapeDtypeStruct((1, 1), jnp.float32),
        compiler_params=pltpu.CompilerParams(
            dimension_semantics=("arbitrary",)),
    )(x, zq)


def kernel(z_e, embed):
    B, M, D = z_e.shape
    N = B * M
    K = embed.shape[0]
    x = z_e.reshape(N, D)
    eTm2 = embed.T * (-2.0)

    idx_col, perp, util = _argmin_call(x, eTm2, N, D, K)
    indices = idx_col.reshape(N)

    zq_flat = _make_sc_gather(N, D, chunk=128)(embed, indices)
    loss = _loss_call(x, zq_flat, N, D)

    return (zq_flat.reshape(B, M, D), idx_col.reshape(B, M),
            loss[0, 0], perp[0, 0], util[0, 0])


# speedup vs baseline: 3.0510x; 1.3241x over previous
import functools

import jax
import jax.numpy as jnp
from jax import lax
from jax.experimental import pallas as pl
from jax.experimental.pallas import tpu as pltpu
from jax.experimental.pallas import tpu_sc as plsc

COMMIT_W = 0.25


def _argmin_body(x_ref, eTm2_ref, idx_ref, e2_ref, iota_ref, *, k_codes):
    r = pl.program_id(0)

    @pl.when(r == 0)
    def _init():
        eTm2 = eTm2_ref[...]
        e2_ref[...] = 0.25 * jnp.sum(eTm2 * eTm2, axis=0, keepdims=True)
        iota_ref[...] = lax.broadcasted_iota(
            jnp.int32, iota_ref.shape, 1).astype(jnp.float32)

    x = x_ref[...]
    raw = lax.dot_general(
        x, eTm2_ref[...], (((1,), (0,)), ((), ())),
        precision=lax.Precision.DEFAULT,
        preferred_element_type=jnp.float32)
    scores = raw + e2_ref[...]
    m = jnp.min(scores, axis=1, keepdims=True)
    cand = jnp.where(scores == m, iota_ref[...], float(k_codes))
    idx_ref[...] = jnp.min(cand, axis=1).astype(jnp.int32)


def _argmin_call(x, eTm2, n_rows, d, k_codes):
    BR = 256
    G = n_rows // BR
    return pl.pallas_call(
        functools.partial(_argmin_body, k_codes=k_codes),
        grid=(G,),
        in_specs=[
            pl.BlockSpec((BR, d), lambda r: (r, 0)),
            pl.BlockSpec((d, k_codes), lambda r: (0, 0)),
        ],
        out_specs=pl.BlockSpec((BR,), lambda r: (r,)),
        out_shape=jax.ShapeDtypeStruct((n_rows,), jnp.int32),
        scratch_shapes=[
            pltpu.VMEM((1, k_codes), jnp.float32),
            pltpu.VMEM((1, k_codes), jnp.float32),
        ],
        compiler_params=pltpu.CompilerParams(
            dimension_semantics=("arbitrary",)),
    )(x, eTm2)


def _make_sc_gather(n_rows, d, k_codes, chunk):
    info = plsc.get_sparse_core_info()
    nc, ns, nl = info.num_cores, info.num_subcores, info.num_lanes
    nw = nc * ns
    rows_per_w = n_rows // nw
    n_chunks = rows_per_w // chunk
    mesh = plsc.VectorSubcoreMesh(core_axis_name="c", subcore_axis_name="s")

    @functools.partial(
        pl.kernel, mesh=mesh,
        out_type=(
            jax.ShapeDtypeStruct((n_rows, d), jnp.float32),
            jax.ShapeDtypeStruct((nc, k_codes), jnp.float32),
        ),
        scratch_types=[
            pltpu.VMEM((chunk,), jnp.int32),
            pltpu.VMEM((chunk, d), jnp.float32),
            pltpu.VMEM((chunk,), jnp.float32),
            pltpu.VMEM_SHARED((k_codes,), jnp.float32),
            pltpu.SemaphoreType.DMA,
        ],
    )
    def gather(table_hbm, idx_hbm, zeros_hbm, zq_hbm, cnt_hbm,
               idx_v, rows_v, ones_v, shared, sem):
        cid = lax.axis_index("c")
        sid = lax.axis_index("s")
        wid = sid * nc + cid
        base = wid * rows_per_w

        for v in range(chunk // nl):
            ones_v[pl.ds(v * nl, nl)] = jnp.ones((nl,), jnp.float32)

        @pl.when(sid == 0)
        def _zero():
            pltpu.sync_copy(zeros_hbm, shared)

        plsc.subcore_barrier()
        for c in range(n_chunks):
            off = base + c * chunk
            pltpu.sync_copy(idx_hbm.at[pl.ds(off, chunk)], idx_v)
            pltpu.async_copy(table_hbm.at[idx_v], rows_v, sem).wait()
            pltpu.sync_copy(rows_v, zq_hbm.at[pl.ds(off, chunk)])
            pltpu.sync_copy(ones_v, shared.at[idx_v], add=True)
        plsc.subcore_barrier()

        @pl.when(sid == 0)
        def _dump():
            pltpu.sync_copy(shared, cnt_hbm.at[cid])

    return gather


def _epi_body(x_ref, zq_ref, cnt_ref, loss_ref, perp_ref, util_ref,
              *, n_rows, k_codes, n_blocks, scale):
    r = pl.program_id(0)

    @pl.when(r == 0)
    def _init():
        loss_ref[...] = jnp.zeros_like(loss_ref)

    diff = zq_ref[...] - x_ref[...]
    loss_ref[...] += jnp.sum(diff * diff, keepdims=True).reshape(1, 1)

    @pl.when(r == n_blocks - 1)
    def _finalize():
        loss_ref[...] = loss_ref[...] * scale
        p = jnp.sum(cnt_ref[...], axis=0, keepdims=True) / n_rows + 1e-10
        perp_ref[...] = jnp.exp(
            -jnp.sum(p * jnp.log(p), keepdims=True).reshape(1, 1))
        util_ref[...] = jnp.sum((p > 1e-8).astype(jnp.float32),
                                keepdims=True).reshape(1, 1) / k_codes


def _epi_call(x, zq, cnt, n_rows, d, k_codes):
    BR = 1024
    G = n_rows // BR
    nc = cnt.shape[0]
    scale = (1.0 + COMMIT_W) / (n_rows * d)
    scalar_spec = pl.BlockSpec((1, 1), lambda r: (0, 0))
    return pl.pallas_call(
        functools.partial(_epi_body, n_rows=n_rows, k_codes=k_codes,
                          n_blocks=G, scale=scale),
        grid=(G,),
        in_specs=[
            pl.BlockSpec((BR, d), lambda r: (r, 0)),
            pl.BlockSpec((BR, d), lambda r: (r, 0)),
            pl.BlockSpec((nc, k_codes), lambda r: (0, 0)),
        ],
        out_specs=(scalar_spec, scalar_spec, scalar_spec),
        out_shape=(
            jax.ShapeDtypeStruct((1, 1), jnp.float32),
            jax.ShapeDtypeStruct((1, 1), jnp.float32),
            jax.ShapeDtypeStruct((1, 1), jnp.float32),
        ),
        compiler_params=pltpu.CompilerParams(
            dimension_semantics=("arbitrary",)),
    )(x, zq, cnt)


def kernel(z_e, embed):
    B, M, D = z_e.shape
    N = B * M
    K = embed.shape[0]
    x = z_e.reshape(N, D)
    eTm2 = embed.T * (-2.0)
    zeros_k = jnp.zeros((K,), jnp.float32)

    indices = _argmin_call(x, eTm2, N, D, K)
    zq_flat, cnt = _make_sc_gather(N, D, K, chunk=128)(embed, indices, zeros_k)
    loss, perp, util = _epi_call(x, zq_flat, cnt, N, D, K)

    return (zq_flat.reshape(B, M, D), indices.reshape(B, M),
            loss[0, 0], perp[0, 0], util[0, 0])
